# Initial kernel scaffold; baseline (speedup 1.0000x reference)
import jax
import jax.numpy as jnp
from jax.experimental import pallas as pl


def kernel(unknown, known, unknow_feats, known_feats, W1, g1, b1, W2, g2, b2):
    raise NotImplementedError("write your pallas kernel here")


# speedup vs baseline: 23.4280x; 23.4280x over previous
import jax
import jax.numpy as jnp
from jax.experimental import pallas as pl


def _nn_mlp1_body(known_ref, unkT_ref, kf_ref, uf_ref, W1_ref,
                  y1_ref, stats_ref):
    b = pl.program_id(0)
    t = pl.program_id(1)
    K = known_ref[0]
    Ut = unkT_ref[0]
    M = K.shape[0]
    mm = jnp.dot(K, Ut, preferred_element_type=jnp.float32)
    d = -2.0 * mm
    d = d + Ut[4:5, :]
    d = d + K[:, 3:4]
    row = jax.lax.broadcasted_iota(jnp.int32, d.shape, 0)
    dd = d
    recips = []
    idxs = []
    for _ in range(3):
        mn = jnp.min(dd, axis=0, keepdims=True)
        amn = jnp.min(jnp.where(dd == mn, row, M), axis=0, keepdims=True)
        recips.append(1.0 / (mn + 1e-8))
        idxs.append(amn)
        dd = jnp.where(row == amn, jnp.float32(jnp.inf), dd)
    norm = recips[0] + recips[1] + recips[2]
    wmatT = jnp.where(row == idxs[0], recips[0] / norm, 0.0)
    wmatT = wmatT + jnp.where(row == idxs[1], recips[1] / norm, 0.0)
    wmatT = wmatT + jnp.where(row == idxs[2], recips[2] / norm, 0.0)
    interpT = jax.lax.dot_general(
        kf_ref[0], wmatT, (((1,), (0,)), ((), ())),
        precision=jax.lax.Precision.HIGHEST,
        preferred_element_type=jnp.float32)
    feat = jnp.concatenate([interpT, uf_ref[0]], axis=0)
    y1 = jnp.dot(W1_ref[...], feat, preferred_element_type=jnp.float32)

    @pl.when((b == 0) & (t == 0))
    def _init():
        stats_ref[...] = jnp.zeros_like(stats_ref)

    y1_ref[0] = y1
    s1 = jnp.sum(y1, axis=1, keepdims=True)
    s2 = jnp.sum(y1 * y1, axis=1, keepdims=True)
    stats_ref[...] = stats_ref[...] + jnp.concatenate([s1, s2], axis=1)


def _mlp2_body(y1_ref, ss1_ref, W2_ref, y2_ref, stats_ref):
    b = pl.program_id(0)
    t = pl.program_id(1)
    sc = ss1_ref[:, 0:1]
    sh = ss1_ref[:, 1:2]
    z = jnp.maximum(y1_ref[0] * sc + sh, 0.0)
    y2 = jnp.dot(W2_ref[...], z, preferred_element_type=jnp.float32)

    @pl.when((b == 0) & (t == 0))
    def _init():
        stats_ref[...] = jnp.zeros_like(stats_ref)

    y2_ref[0] = y2
    s1 = jnp.sum(y2, axis=1, keepdims=True)
    s2 = jnp.sum(y2 * y2, axis=1, keepdims=True)
    stats_ref[...] = stats_ref[...] + jnp.concatenate([s1, s2], axis=1)


def _bn_relu_body(y2_ref, ss2_ref, out_ref):
    sc = ss2_ref[:, 0:1]
    sh = ss2_ref[:, 1:2]
    out_ref[0] = jnp.maximum(y2_ref[0] * sc + sh, 0.0)


def kernel(unknown, known, unknow_feats, known_feats, W1, g1, b1, W2, g2, b2):
    B, N, _ = unknown.shape
    M = known.shape[1]
    C1 = unknow_feats.shape[1]
    C2 = known_feats.shape[1]
    CO1 = W1.shape[0]
    CO2 = W2.shape[0]
    NT = 512
    nT = N // NT

    uu = jnp.sum(unknown ** 2, axis=-1)
    kk = jnp.sum(known ** 2, axis=-1)
    z1 = jnp.zeros((B, 1, N), jnp.float32)
    unkT = jnp.concatenate(
        [jnp.transpose(unknown, (0, 2, 1)), z1, uu[:, None, :], z1, z1, z1],
        axis=1)
    z2 = jnp.zeros((B, M, 1), jnp.float32)
    knownP = jnp.concatenate(
        [known, kk[:, :, None], z2, z2, z2, z2], axis=2)

    y1, stats1 = pl.pallas_call(
        _nn_mlp1_body,
        grid=(B, nT),
        in_specs=[
            pl.BlockSpec((1, M, 8), lambda b, t: (b, 0, 0)),
            pl.BlockSpec((1, 8, NT), lambda b, t: (b, 0, t)),
            pl.BlockSpec((1, C2, M), lambda b, t: (b, 0, 0)),
            pl.BlockSpec((1, C1, NT), lambda b, t: (b, 0, t)),
            pl.BlockSpec((CO1, C1 + C2), lambda b, t: (0, 0)),
        ],
        out_specs=[
            pl.BlockSpec((1, CO1, NT), lambda b, t: (b, 0, t)),
            pl.BlockSpec((CO1, 2), lambda b, t: (0, 0)),
        ],
        out_shape=[
            jax.ShapeDtypeStruct((B, CO1, N), jnp.float32),
            jax.ShapeDtypeStruct((CO1, 2), jnp.float32),
        ],
    )(knownP, unkT, known_feats, unknow_feats, W1)

    cnt = float(B * N)
    mean1 = stats1[:, 0] / cnt
    var1 = stats1[:, 1] / cnt - mean1 * mean1
    scale1 = g1 / jnp.sqrt(var1 + 1e-5)
    shift1 = b1 - mean1 * scale1
    ss1 = jnp.stack([scale1, shift1], axis=1)

    y2, stats2 = pl.pallas_call(
        _mlp2_body,
        grid=(B, nT),
        in_specs=[
            pl.BlockSpec((1, CO1, NT), lambda b, t: (b, 0, t)),
            pl.BlockSpec((CO1, 2), lambda b, t: (0, 0)),
            pl.BlockSpec((CO2, CO1), lambda b, t: (0, 0)),
        ],
        out_specs=[
            pl.BlockSpec((1, CO2, NT), lambda b, t: (b, 0, t)),
            pl.BlockSpec((CO2, 2), lambda b, t: (0, 0)),
        ],
        out_shape=[
            jax.ShapeDtypeStruct((B, CO2, N), jnp.float32),
            jax.ShapeDtypeStruct((CO2, 2), jnp.float32),
        ],
    )(y1, ss1, W2)

    mean2 = stats2[:, 0] / cnt
    var2 = stats2[:, 1] / cnt - mean2 * mean2
    scale2 = g2 / jnp.sqrt(var2 + 1e-5)
    shift2 = b2 - mean2 * scale2
    ss2 = jnp.stack([scale2, shift2], axis=1)

    out = pl.pallas_call(
        _bn_relu_body,
        grid=(B, nT),
        in_specs=[
            pl.BlockSpec((1, CO2, NT), lambda b, t: (b, 0, t)),
            pl.BlockSpec((CO2, 2), lambda b, t: (0, 0)),
        ],
        out_specs=pl.BlockSpec((1, CO2, NT), lambda b, t: (b, 0, t)),
        out_shape=jax.ShapeDtypeStruct((B, CO2, N), jnp.float32),
    )(y2, ss2)
    return out
